# baseline (speedup 1.0000x reference)
import jax
import jax.numpy as jnp
from jax import lax
from jax.experimental import pallas as pl
from jax.experimental.pallas import tpu as pltpu
from jax.experimental.pallas import tpu_sc as plsc

_N = 10000
_E = 320000
_D = 128

_CH = 128
_G_REAL = _E // _CH
_G_PAD = 2560
_E_PAD = _G_PAD * _CH
_N_PAD = 10240
_ZROWS = _N_PAD // 16
_OROWS = 624
_TAIL0 = 16 * _OROWS
_NC = 2
_NS = 16
_CHA = _G_PAD // _NS
_CHB = _G_PAD // (_NC * _NS)
_BQ = 16
_BQE = 32
_ROWS_TC = 1000

_mesh = plsc.VectorSubcoreMesh(core_axis_name="c", subcore_axis_name="s")


def _zero_vmem_rows(buf):
    zeros16 = jnp.zeros((16,), jnp.float32)

    @pl.loop(0, _CH)
    def _zrow(i):
        for k in range(_D // 16):
            buf[i, pl.ds(k * 16, 16)] = zeros16


def _zero_spmem_slab(src_v, acc_sh, s):
    z0 = s * _ZROWS
    for k in range(_ZROWS // _CH):
        pltpu.sync_copy(src_v, acc_sh.at[pl.ds(z0 + k * _CH, _CH)])


def _copy_out_rows(acc_sh, out_ref, s):
    r0 = s * _OROWS
    pltpu.sync_copy(acc_sh.at[pl.ds(r0, _OROWS)], out_ref.at[pl.ds(r0, _OROWS)])

    @pl.when(s == _NS - 1)
    def _():
        pltpu.sync_copy(acc_sh.at[pl.ds(_TAIL0, _N - _TAIL0)],
                        out_ref.at[pl.ds(_TAIL0, _N - _TAIL0)])


def _segsum_ef_body(ef_hbm, dst2_hbm, se_out,
                    ef0_v, ef1_v, didx_v, acc_sh, semg0, semg1, sems0, sems1):
    c = lax.axis_index("c")
    s = lax.axis_index("s")

    _zero_vmem_rows(ef0_v)
    _zero_spmem_slab(ef0_v, acc_sh, s)

    plsc.subcore_barrier()

    nch = jnp.minimum(_CHA, _G_REAL - s * _CHA)
    nbq = (nch + _BQE - 1) // _BQE
    g0 = s * _CHA

    def _rd(g, buf, sem):
        return pltpu.async_copy(
            ef_hbm.at[pl.ds(g * _CH, _CH), pl.ds(c * _D, _D)], buf, sem)

    def _rd_wait(g, buf, sem):
        pltpu.make_async_copy(
            ef_hbm.at[pl.ds(g * _CH, _CH), pl.ds(c * _D, _D)], buf, sem).wait()

    _rd(g0, ef0_v, semg0)
    _rd(g0 + 1, ef1_v, semg1)

    @pl.loop(0, nbq)
    def _block(q):
        b0 = g0 + q * _BQE
        pltpu.sync_copy(dst2_hbm.at[pl.ds(q * _BQE + s * _CHA, _BQE)], didx_v)
        for t in range(_BQE // 2):
            j0 = 2 * t

            @pl.when(q * _BQE + j0 < nch)
            def _():
                _rd_wait(b0 + j0, ef0_v, semg0)
                pltpu.async_copy(ef0_v, acc_sh.at[didx_v.at[j0]], sems0,
                                 add=True)
                _rd_wait(b0 + j0 + 1, ef1_v, semg1)
                pltpu.async_copy(ef1_v, acc_sh.at[didx_v.at[j0 + 1]], sems1,
                                 add=True)
                pltpu.make_async_copy(ef0_v, acc_sh.at[didx_v.at[j0]],
                                      sems0).wait()

                @pl.when(q * _BQE + j0 + 2 < nch)
                def _():
                    _rd(b0 + j0 + 2, ef0_v, semg0)

                pltpu.make_async_copy(ef1_v, acc_sh.at[didx_v.at[j0 + 1]],
                                      sems1).wait()

                @pl.when(q * _BQE + j0 + 3 < nch)
                def _():
                    _rd(b0 + j0 + 3, ef1_v, semg1)

    plsc.subcore_barrier()
    _copy_out_rows(acc_sh, se_out.at[c], s)


_segsum_ef = pl.kernel(
    _segsum_ef_body,
    out_type=jax.ShapeDtypeStruct((_NC, _N, _D), jnp.float32),
    mesh=_mesh,
    scratch_types=[
        pltpu.VMEM((_CH, _D), jnp.float32),
        pltpu.VMEM((_CH, _D), jnp.float32),
        pltpu.VMEM((_BQE, _CH), jnp.int32),
        pltpu.VMEM_SHARED((_N_PAD, _D), jnp.float32),
        pltpu.SemaphoreType.DMA,
        pltpu.SemaphoreType.DMA,
        pltpu.SemaphoreType.DMA,
        pltpu.SemaphoreType.DMA,
    ],
)


def _make_segsum_rows(with_deg):
    def body(h_hbm, src2_hbm, dst2_hbm, *refs):
        if with_deg:
            (g_out, deg_out, sidx_v, didx_v, rows0_v, rows1_v, ones_v,
             degv_v, acc_sh, deg_sh, sem0, sem1) = refs
        else:
            (g_out, sidx_v, didx_v, rows0_v, rows1_v,
             acc_sh, sem0, sem1) = refs
        c = lax.axis_index("c")
        s = lax.axis_index("s")
        w = s * _NC + c

        _zero_vmem_rows(rows0_v)
        _zero_spmem_slab(rows0_v, acc_sh, s)
        if with_deg:
            ones16 = jnp.ones((16,), jnp.float32)
            for k in range(_CH // 16):
                ones_v[pl.ds(k * 16, 16)] = ones16
            z0 = s * _ZROWS
            for k in range(_ZROWS // _CH):
                pltpu.sync_copy(rows0_v.at[0], deg_sh.at[pl.ds(z0 + k * _CH, _CH)])
        plsc.subcore_barrier()

        @pl.loop(0, _CHB // _BQ)
        def _block(q):
            b0 = w * _CHB + q * _BQ
            pltpu.sync_copy(src2_hbm.at[pl.ds(b0, _BQ)], sidx_v)
            pltpu.sync_copy(dst2_hbm.at[pl.ds(b0, _BQ)], didx_v)
            pltpu.async_copy(h_hbm.at[sidx_v.at[0]], rows0_v, sem0)
            for t in range(_BQ // 2):
                j0 = 2 * t
                pltpu.make_async_copy(
                    h_hbm.at[sidx_v.at[j0]], rows0_v, sem0).wait()
                pltpu.async_copy(h_hbm.at[sidx_v.at[j0 + 1]], rows1_v, sem1)
                pltpu.sync_copy(rows0_v, acc_sh.at[didx_v.at[j0]], add=True)
                if with_deg:
                    pltpu.sync_copy(ones_v, deg_sh.at[didx_v.at[j0]], add=True)
                pltpu.make_async_copy(
                    h_hbm.at[sidx_v.at[j0 + 1]], rows1_v, sem1).wait()
                if j0 + 2 < _BQ:
                    pltpu.async_copy(h_hbm.at[sidx_v.at[j0 + 2]], rows0_v, sem0)
                pltpu.sync_copy(rows1_v, acc_sh.at[didx_v.at[j0 + 1]], add=True)
                if with_deg:
                    pltpu.sync_copy(ones_v, deg_sh.at[didx_v.at[j0 + 1]],
                                    add=True)

        plsc.subcore_barrier()
        _copy_out_rows(acc_sh, g_out.at[c], s)
        if with_deg:
            r0 = s * _OROWS
            pltpu.sync_copy(deg_sh.at[pl.ds(r0, _OROWS)],
                            degv_v.at[pl.ds(0, _OROWS)])
            pltpu.sync_copy(degv_v.at[pl.ds(0, _OROWS)],
                            deg_out.at[pl.ds(c * _N + r0, _OROWS)])

            @pl.when(s == _NS - 1)
            def _():
                pltpu.sync_copy(deg_sh.at[pl.ds(_TAIL0, _N - _TAIL0)],
                                degv_v.at[pl.ds(0, _N - _TAIL0)])
                pltpu.sync_copy(degv_v.at[pl.ds(0, _N - _TAIL0)],
                                deg_out.at[pl.ds(c * _N + _TAIL0, _N - _TAIL0)])

    out_type = jax.ShapeDtypeStruct((_NC, _N, _D), jnp.float32)
    scratch = [
        pltpu.VMEM((_BQ, _CH), jnp.int32),
        pltpu.VMEM((_BQ, _CH), jnp.int32),
        pltpu.VMEM((_CH, _D), jnp.float32),
        pltpu.VMEM((_CH, _D), jnp.float32),
    ]
    if with_deg:
        out_type = (out_type, jax.ShapeDtypeStruct((_NC * _N,), jnp.float32))
        scratch = scratch + [pltpu.VMEM((_CH,), jnp.float32),
                             pltpu.VMEM((_OROWS + 16,), jnp.float32)]
    scratch = scratch + [pltpu.VMEM_SHARED((_N_PAD, _D), jnp.float32)]
    if with_deg:
        scratch = scratch + [pltpu.VMEM_SHARED((_N_PAD,), jnp.float32)]
    scratch = scratch + [pltpu.SemaphoreType.DMA, pltpu.SemaphoreType.DMA]
    return pl.kernel(body, out_type=out_type, mesh=_mesh,
                     scratch_types=scratch)


_segsum_rows_deg = _make_segsum_rows(True)
_segsum_rows = _make_segsum_rows(False)


def _dense_body(g_ref, se_ref, degp_ref, h_ref, wa_ref, wb_ref, b_ref, out_ref):
    gsum = g_ref[0] + g_ref[1]
    acc = jnp.dot(gsum, wa_ref[...], preferred_element_type=jnp.float32)
    acc = acc + jnp.dot(se_ref[0], wb_ref[0], preferred_element_type=jnp.float32)
    acc = acc + jnp.dot(se_ref[1], wb_ref[1], preferred_element_type=jnp.float32)
    deg = degp_ref[0] + degp_ref[1]
    acc = acc + deg * b_ref[...]
    rdeg = 1.0 / jnp.maximum(deg, 1.0)
    out_ref[...] = jnp.tanh(acc * rdeg + h_ref[...])


def _dense(gparts, se2, degp, h, w, b):
    wa = w[:_D]
    wb = w[_D:].reshape(_NC, _D, _D)
    b2 = b.reshape(1, _D)
    r = _ROWS_TC
    return pl.pallas_call(
        _dense_body,
        grid=(_N // r,),
        in_specs=[
            pl.BlockSpec((_NC, r, _D), lambda i: (0, i, 0)),
            pl.BlockSpec((_NC, r, _D), lambda i: (0, i, 0)),
            pl.BlockSpec((_NC, r, 1), lambda i: (0, i, 0)),
            pl.BlockSpec((r, _D), lambda i: (i, 0)),
            pl.BlockSpec((_D, _D), lambda i: (0, 0)),
            pl.BlockSpec((_NC, _D, _D), lambda i: (0, 0, 0)),
            pl.BlockSpec((1, _D), lambda i: (0, 0)),
        ],
        out_specs=pl.BlockSpec((r, _D), lambda i: (i, 0)),
        out_shape=jax.ShapeDtypeStruct((_N, _D), jnp.float32),
    )(gparts, se2, degp, h, wa, wb, b2)


@jax.jit
def kernel(node_feats, edge_feats, edge_index, edge_types, W1_0, b1_0, W1_1, b1_1):
    del edge_types
    src = edge_index[0].astype(jnp.int32)
    dst = edge_index[1].astype(jnp.int32)
    pad = _E_PAD - _E
    padr = jnp.arange(pad, dtype=jnp.int32)
    trash = _N + padr % (_N_PAD - _N)
    src2 = jnp.concatenate([src, padr % _N]).reshape(_G_PAD, _CH)
    dst2 = jnp.concatenate([dst, trash]).reshape(_G_PAD, _CH)

    se2 = _segsum_ef(edge_feats, dst2)
    g1, degf = _segsum_rows_deg(node_feats, src2, dst2)
    degp = degf.reshape(_NC, _N, 1)
    h1 = _dense(g1, se2, degp, node_feats, W1_0, b1_0)
    g2 = _segsum_rows(h1, src2, dst2)
    h2 = _dense(g2, se2, degp, h1, W1_1, b1_1)
    return h2


# speedup vs baseline: 1.0663x; 1.0663x over previous
import jax
import jax.numpy as jnp
from jax import lax
from jax.experimental import pallas as pl
from jax.experimental.pallas import tpu as pltpu
from jax.experimental.pallas import tpu_sc as plsc

_N = 10000
_E = 320000
_D = 128

_CH = 128
_G_REAL = _E // _CH
_G_PAD = 2560
_E_PAD = _G_PAD * _CH
_N_PAD = 10240
_ZROWS = _N_PAD // 16
_OROWS = 624
_TAIL0 = 16 * _OROWS
_NC = 2
_NS = 16
_CHA = _G_PAD // _NS
_CHB = _G_PAD // (_NC * _NS)
_BQ = 16
_BQE = 32
_ROWS_TC = 1000

_mesh = plsc.VectorSubcoreMesh(core_axis_name="c", subcore_axis_name="s")


def _zero_vmem_rows(buf):
    zeros16 = jnp.zeros((16,), jnp.float32)

    @pl.loop(0, _CH)
    def _zrow(i):
        for k in range(_D // 16):
            buf[i, pl.ds(k * 16, 16)] = zeros16


def _zero_spmem_slab(src_v, acc_sh, s):
    z0 = s * _ZROWS
    for k in range(_ZROWS // _CH):
        pltpu.sync_copy(src_v, acc_sh.at[pl.ds(z0 + k * _CH, _CH)])


def _copy_out_rows(acc_sh, out_ref, s):
    r0 = s * _OROWS
    pltpu.sync_copy(acc_sh.at[pl.ds(r0, _OROWS)], out_ref.at[pl.ds(r0, _OROWS)])

    @pl.when(s == _NS - 1)
    def _():
        pltpu.sync_copy(acc_sh.at[pl.ds(_TAIL0, _N - _TAIL0)],
                        out_ref.at[pl.ds(_TAIL0, _N - _TAIL0)])


def _segsum_ef_body(ef_hbm, dst2_hbm, se_out,
                    ef0_v, ef1_v, didx_v, acc_sh, sem0, sem1):
    c = lax.axis_index("c")
    s = lax.axis_index("s")

    _zero_vmem_rows(ef0_v)
    _zero_spmem_slab(ef0_v, acc_sh, s)

    plsc.subcore_barrier()

    nch = jnp.minimum(_CHA, _G_REAL - s * _CHA)
    nbq = (nch + _BQE - 1) // _BQE

    def _rd(g, buf, sem):
        return pltpu.async_copy(
            ef_hbm.at[pl.ds(g * _CH, _CH), pl.ds(c * _D, _D)], buf, sem)

    def _rd_wait(g, buf, sem):
        pltpu.make_async_copy(
            ef_hbm.at[pl.ds(g * _CH, _CH), pl.ds(c * _D, _D)], buf, sem).wait()

    @pl.loop(0, nbq)
    def _block(q):
        b0 = s * _CHA + q * _BQE
        _rd(b0, ef0_v, sem0)
        pltpu.sync_copy(dst2_hbm.at[pl.ds(b0, _BQE)], didx_v)
        for t in range(_BQE // 2):
            j0 = 2 * t

            @pl.when(q * _BQE + j0 < nch)
            def _():
                _rd_wait(b0 + j0, ef0_v, sem0)
                _rd(b0 + j0 + 1, ef1_v, sem1)
                pltpu.sync_copy(ef0_v, acc_sh.at[didx_v.at[j0]], add=True)
                _rd_wait(b0 + j0 + 1, ef1_v, sem1)

                if j0 + 2 < _BQE:
                    @pl.when(q * _BQE + j0 + 2 < nch)
                    def _():
                        _rd(b0 + j0 + 2, ef0_v, sem0)

                pltpu.sync_copy(ef1_v, acc_sh.at[didx_v.at[j0 + 1]], add=True)

    plsc.subcore_barrier()
    _copy_out_rows(acc_sh, se_out.at[c], s)


_segsum_ef = pl.kernel(
    _segsum_ef_body,
    out_type=jax.ShapeDtypeStruct((_NC, _N, _D), jnp.float32),
    mesh=_mesh,
    scratch_types=[
        pltpu.VMEM((_CH, _D), jnp.float32),
        pltpu.VMEM((_CH, _D), jnp.float32),
        pltpu.VMEM((_BQE, _CH), jnp.int32),
        pltpu.VMEM_SHARED((_N_PAD, _D), jnp.float32),
        pltpu.SemaphoreType.DMA,
        pltpu.SemaphoreType.DMA,
    ],
)


def _make_segsum_rows(with_deg):
    def body(h_hbm, src2_hbm, dst2_hbm, *refs):
        if with_deg:
            (g_out, deg_out, sidx_v, didx_v, rows0_v, rows1_v, ones_v,
             degv_v, acc_sh, deg_sh, sem0, sem1) = refs
        else:
            (g_out, sidx_v, didx_v, rows0_v, rows1_v,
             acc_sh, sem0, sem1) = refs
        c = lax.axis_index("c")
        s = lax.axis_index("s")
        w = s * _NC + c

        _zero_vmem_rows(rows0_v)
        _zero_spmem_slab(rows0_v, acc_sh, s)
        if with_deg:
            ones16 = jnp.ones((16,), jnp.float32)
            for k in range(_CH // 16):
                ones_v[pl.ds(k * 16, 16)] = ones16
            z0 = s * _ZROWS
            for k in range(_ZROWS // _CH):
                pltpu.sync_copy(rows0_v.at[0], deg_sh.at[pl.ds(z0 + k * _CH, _CH)])
        plsc.subcore_barrier()

        @pl.loop(0, _CHB // _BQ)
        def _block(q):
            b0 = w * _CHB + q * _BQ
            pltpu.sync_copy(src2_hbm.at[pl.ds(b0, _BQ)], sidx_v)
            pltpu.sync_copy(dst2_hbm.at[pl.ds(b0, _BQ)], didx_v)
            pltpu.async_copy(h_hbm.at[sidx_v.at[0]], rows0_v, sem0)
            for t in range(_BQ // 2):
                j0 = 2 * t
                pltpu.make_async_copy(
                    h_hbm.at[sidx_v.at[j0]], rows0_v, sem0).wait()
                pltpu.async_copy(h_hbm.at[sidx_v.at[j0 + 1]], rows1_v, sem1)
                pltpu.sync_copy(rows0_v, acc_sh.at[didx_v.at[j0]], add=True)
                if with_deg:
                    pltpu.sync_copy(ones_v, deg_sh.at[didx_v.at[j0]], add=True)
                pltpu.make_async_copy(
                    h_hbm.at[sidx_v.at[j0 + 1]], rows1_v, sem1).wait()
                if j0 + 2 < _BQ:
                    pltpu.async_copy(h_hbm.at[sidx_v.at[j0 + 2]], rows0_v, sem0)
                pltpu.sync_copy(rows1_v, acc_sh.at[didx_v.at[j0 + 1]], add=True)
                if with_deg:
                    pltpu.sync_copy(ones_v, deg_sh.at[didx_v.at[j0 + 1]],
                                    add=True)

        plsc.subcore_barrier()
        _copy_out_rows(acc_sh, g_out.at[c], s)
        if with_deg:
            r0 = s * _OROWS
            pltpu.sync_copy(deg_sh.at[pl.ds(r0, _OROWS)],
                            degv_v.at[pl.ds(0, _OROWS)])
            pltpu.sync_copy(degv_v.at[pl.ds(0, _OROWS)],
                            deg_out.at[pl.ds(c * _N + r0, _OROWS)])

            @pl.when(s == _NS - 1)
            def _():
                pltpu.sync_copy(deg_sh.at[pl.ds(_TAIL0, _N - _TAIL0)],
                                degv_v.at[pl.ds(0, _N - _TAIL0)])
                pltpu.sync_copy(degv_v.at[pl.ds(0, _N - _TAIL0)],
                                deg_out.at[pl.ds(c * _N + _TAIL0, _N - _TAIL0)])

    out_type = jax.ShapeDtypeStruct((_NC, _N, _D), jnp.float32)
    scratch = [
        pltpu.VMEM((_BQ, _CH), jnp.int32),
        pltpu.VMEM((_BQ, _CH), jnp.int32),
        pltpu.VMEM((_CH, _D), jnp.float32),
        pltpu.VMEM((_CH, _D), jnp.float32),
    ]
    if with_deg:
        out_type = (out_type, jax.ShapeDtypeStruct((_NC * _N,), jnp.float32))
        scratch = scratch + [pltpu.VMEM((_CH,), jnp.float32),
                             pltpu.VMEM((_OROWS + 16,), jnp.float32)]
    scratch = scratch + [pltpu.VMEM_SHARED((_N_PAD, _D), jnp.float32)]
    if with_deg:
        scratch = scratch + [pltpu.VMEM_SHARED((_N_PAD,), jnp.float32)]
    scratch = scratch + [pltpu.SemaphoreType.DMA, pltpu.SemaphoreType.DMA]
    return pl.kernel(body, out_type=out_type, mesh=_mesh,
                     scratch_types=scratch)


_segsum_rows_deg = _make_segsum_rows(True)
_segsum_rows = _make_segsum_rows(False)


def _dense_body(g_ref, se_ref, degp_ref, h_ref, wa_ref, wb_ref, b_ref, out_ref):
    gsum = g_ref[0] + g_ref[1]
    acc = jnp.dot(gsum, wa_ref[...], preferred_element_type=jnp.float32)
    acc = acc + jnp.dot(se_ref[0], wb_ref[0], preferred_element_type=jnp.float32)
    acc = acc + jnp.dot(se_ref[1], wb_ref[1], preferred_element_type=jnp.float32)
    deg = degp_ref[0] + degp_ref[1]
    acc = acc + deg * b_ref[...]
    rdeg = 1.0 / jnp.maximum(deg, 1.0)
    out_ref[...] = jnp.tanh(acc * rdeg + h_ref[...])


def _dense(gparts, se2, degp, h, w, b):
    wa = w[:_D]
    wb = w[_D:].reshape(_NC, _D, _D)
    b2 = b.reshape(1, _D)
    r = _ROWS_TC
    return pl.pallas_call(
        _dense_body,
        grid=(_N // r,),
        in_specs=[
            pl.BlockSpec((_NC, r, _D), lambda i: (0, i, 0)),
            pl.BlockSpec((_NC, r, _D), lambda i: (0, i, 0)),
            pl.BlockSpec((_NC, r, 1), lambda i: (0, i, 0)),
            pl.BlockSpec((r, _D), lambda i: (i, 0)),
            pl.BlockSpec((_D, _D), lambda i: (0, 0)),
            pl.BlockSpec((_NC, _D, _D), lambda i: (0, 0, 0)),
            pl.BlockSpec((1, _D), lambda i: (0, 0)),
        ],
        out_specs=pl.BlockSpec((r, _D), lambda i: (i, 0)),
        out_shape=jax.ShapeDtypeStruct((_N, _D), jnp.float32),
    )(gparts, se2, degp, h, wa, wb, b2)


@jax.jit
def kernel(node_feats, edge_feats, edge_index, edge_types, W1_0, b1_0, W1_1, b1_1):
    del edge_types
    src = edge_index[0].astype(jnp.int32)
    dst = edge_index[1].astype(jnp.int32)
    pad = _E_PAD - _E
    padr = jnp.arange(pad, dtype=jnp.int32)
    trash = _N + padr % (_N_PAD - _N)
    src2 = jnp.concatenate([src, padr % _N]).reshape(_G_PAD, _CH)
    dst2 = jnp.concatenate([dst, trash]).reshape(_G_PAD, _CH)

    se2 = _segsum_ef(edge_feats, dst2)
    g1, degf = _segsum_rows_deg(node_feats, src2, dst2)
    degp = degf.reshape(_NC, _N, 1)
    h1 = _dense(g1, se2, degp, node_feats, W1_0, b1_0)
    g2 = _segsum_rows(h1, src2, dst2)
    h2 = _dense(g2, se2, degp, h1, W1_1, b1_1)
    return h2
